# baseline (speedup 1.0000x reference)
import jax
import jax.numpy as jnp
from jax.experimental import pallas as pl
from jax.experimental.pallas import tpu as pltpu

C = 256
NBUF = 8
LA = 6


def _make_body(batch, seq, d):
    nrows = batch * seq
    nchunk = nrows // C

    def body(labels_sm, ctx_any, emb_any, out_any, buf, cls, sem_cls, sem_in, sem_out):
        copies_in = {}
        copies_out = {}

        def issue_in(j):
            slot = j % NBUF
            if j >= NBUF:
                copies_out[j - NBUF].wait()
            cp = pltpu.make_async_copy(
                ctx_any.at[pl.ds(j * C, C)], buf.at[slot], sem_in.at[slot]
            )
            cp.start()
            copies_in[j] = cp

        for j in range(min(LA, nchunk)):
            issue_in(j)

        cls_copies = []
        for b in range(batch):
            cp = pltpu.make_async_copy(emb_any.at[labels_sm[b]], cls.at[b], sem_cls)
            cp.start()
            cls_copies.append(cp)
        for cp in cls_copies:
            cp.wait()

        for i in range(nchunk):
            slot = i % NBUF
            copies_in[i].wait()
            b = (i * C) // seq
            buf[slot] = buf[slot] + cls[b]
            cp = pltpu.make_async_copy(
                buf.at[slot], out_any.at[pl.ds(i * C, C)], sem_out.at[slot]
            )
            cp.start()
            copies_out[i] = cp
            if i + LA < nchunk:
                issue_in(i + LA)

        for i in range(max(0, nchunk - NBUF), nchunk):
            copies_out[i].wait()

    return body


@jax.jit
def kernel(ctx_vec, labels, emb_weight):
    batch, seq, d = ctx_vec.shape
    flat = ctx_vec.reshape(batch * seq, d)
    out = pl.pallas_call(
        _make_body(batch, seq, d),
        in_specs=[
            pl.BlockSpec(memory_space=pltpu.SMEM),
            pl.BlockSpec(memory_space=pltpu.MemorySpace.HBM),
            pl.BlockSpec(memory_space=pltpu.MemorySpace.HBM),
        ],
        out_specs=pl.BlockSpec(memory_space=pltpu.MemorySpace.HBM),
        out_shape=jax.ShapeDtypeStruct((batch * seq, d), ctx_vec.dtype),
        scratch_shapes=[
            pltpu.VMEM((NBUF, C, d), jnp.float32),
            pltpu.VMEM((batch, d), jnp.float32),
            pltpu.SemaphoreType.DMA,
            pltpu.SemaphoreType.DMA((NBUF,)),
            pltpu.SemaphoreType.DMA((NBUF,)),
        ],
        compiler_params=pltpu.CompilerParams(
            vmem_limit_bytes=60 * 1024 * 1024,
        ),
    )(labels.astype(jnp.int32), flat, emb_weight)
    return out.reshape(batch, seq, d)


# speedup vs baseline: 1.0053x; 1.0053x over previous
import jax
import jax.numpy as jnp
from jax.experimental import pallas as pl
from jax.experimental.pallas import tpu as pltpu

C = 1024
NBUF = 5
LA = 3


def _make_body(batch, seq, d):
    nrows = batch * seq
    nchunk = nrows // C

    def body(labels_sm, ctx_any, emb_any, out_any, buf, cls, sem_cls, sem_in, sem_out):
        copies_in = {}
        copies_out = {}

        def issue_in(j):
            slot = j % NBUF
            if j >= NBUF:
                copies_out[j - NBUF].wait()
            cp = pltpu.make_async_copy(
                ctx_any.at[pl.ds(j * C, C)], buf.at[slot], sem_in.at[slot]
            )
            cp.start()
            copies_in[j] = cp

        for j in range(min(LA, nchunk)):
            issue_in(j)

        cls_copies = []
        for b in range(batch):
            cp = pltpu.make_async_copy(emb_any.at[labels_sm[b]], cls.at[b], sem_cls)
            cp.start()
            cls_copies.append(cp)
        for cp in cls_copies:
            cp.wait()

        for i in range(nchunk):
            slot = i % NBUF
            copies_in[i].wait()
            b = (i * C) // seq
            buf[slot] = buf[slot] + cls[b]
            cp = pltpu.make_async_copy(
                buf.at[slot], out_any.at[pl.ds(i * C, C)], sem_out.at[slot]
            )
            cp.start()
            copies_out[i] = cp
            if i + LA < nchunk:
                issue_in(i + LA)

        for i in range(max(0, nchunk - NBUF), nchunk):
            copies_out[i].wait()

    return body


@jax.jit
def kernel(ctx_vec, labels, emb_weight):
    batch, seq, d = ctx_vec.shape
    flat = ctx_vec.reshape(batch * seq, d)
    out = pl.pallas_call(
        _make_body(batch, seq, d),
        in_specs=[
            pl.BlockSpec(memory_space=pltpu.SMEM),
            pl.BlockSpec(memory_space=pltpu.MemorySpace.HBM),
            pl.BlockSpec(memory_space=pltpu.MemorySpace.HBM),
        ],
        out_specs=pl.BlockSpec(memory_space=pltpu.MemorySpace.HBM),
        out_shape=jax.ShapeDtypeStruct((batch * seq, d), ctx_vec.dtype),
        scratch_shapes=[
            pltpu.VMEM((NBUF, C, d), jnp.float32),
            pltpu.VMEM((batch, d), jnp.float32),
            pltpu.SemaphoreType.DMA,
            pltpu.SemaphoreType.DMA((NBUF,)),
            pltpu.SemaphoreType.DMA((NBUF,)),
        ],
        compiler_params=pltpu.CompilerParams(
            vmem_limit_bytes=60 * 1024 * 1024,
        ),
    )(labels.astype(jnp.int32), flat, emb_weight)
    return out.reshape(batch, seq, d)
